# Initial kernel scaffold; baseline (speedup 1.0000x reference)
import jax
import jax.numpy as jnp
from jax.experimental import pallas as pl


def kernel(input_ids, byte_table, pos_table, ht3, ht4, ht5, proj_W, proj_b, gamma, beta):
    raise NotImplementedError("write your pallas kernel here")


# speedup vs baseline: 6.8994x; 6.8994x over previous
import functools

import jax
import jax.numpy as jnp
from jax import lax
from jax.experimental import pallas as pl
from jax.experimental.pallas import tpu as pltpu
from jax.experimental.pallas import tpu_sc as plsc

_B, _S, _H = 4, 8192, 768
_VOCAB = 260
_HASH_VOCAB = 100000
_HD = 128

_P1 = 257
_P2 = 66049
_P3 = 74593
_P4 = 70401

_NC, _NS = 2, 16
_NW = _NC * _NS
_TPW = (_B * _S) // _NW
_PAD = 8
_CHUNK = 128
_NCHUNK = _TPW // _CHUNK
_GRP = _TPW // 16


def _sc_hash_gather(ids_pad, ht3, ht4, ht5):
    mesh = plsc.VectorSubcoreMesh(core_axis_name="c", subcore_axis_name="s",
                                  num_cores=_NC, num_subcores=_NS)

    @functools.partial(
        pl.kernel,
        out_type=jax.ShapeDtypeStruct((_B * _S, _HD), jnp.float32),
        mesh=mesh,
        scratch_types=[
            pltpu.VMEM((_TPW + _PAD,), jnp.int32),
            pltpu.VMEM((_NCHUNK, _CHUNK), jnp.int32),
            pltpu.VMEM((_NCHUNK, _CHUNK), jnp.int32),
            pltpu.VMEM((_NCHUNK, _CHUNK), jnp.int32),
            pltpu.VMEM((_CHUNK, _HD), jnp.float32),
            pltpu.VMEM((_CHUNK, _HD), jnp.float32),
            pltpu.VMEM((_CHUNK, _HD), jnp.float32),
            pltpu.SemaphoreType.DMA,
        ],
    )
    def body(ids_hbm, t3_hbm, t4_hbm, t5_hbm, out_hbm,
             ids_v, idx3_v, idx4_v, idx5_v, r3_v, r4_v, r5_v, sem):
        wid = lax.axis_index("s") * _NC + lax.axis_index("c")
        wpr = jnp.int32(_S // _TPW)
        b = lax.div(wid, wpr)
        base_in_row = lax.rem(wid, wpr) * _TPW
        start = b * (_S + _PAD) + base_in_row

        pltpu.sync_copy(ids_hbm.at[pl.ds(start, _TPW + _PAD)], ids_v)

        def hash_body(g, _):
            o = g * 16
            vj = ids_v[pl.ds(_PAD + o, 16)]
            v1 = ids_v[pl.ds(_PAD - 1 + o, 16)]
            v2 = ids_v[pl.ds(_PAD - 2 + o, 16)]
            v3 = ids_v[pl.ds(_PAD - 3 + o, 16)]
            v4 = ids_v[pl.ds(_PAD - 4 + o, 16)]
            hv = jnp.int32(_HASH_VOCAB)
            h3 = lax.rem(v2 * _P2 + v1 * _P1 + vj, hv)
            h4 = lax.rem(v3 * _P3 + h3, hv)
            h5 = lax.rem(v4 * _P4 + h4, hv)
            gpc = jnp.int32(_CHUNK // 16)
            c = lax.div(g, gpc)
            lo = lax.rem(g, gpc) * 16
            idx3_v[c, pl.ds(lo, 16)] = h3
            idx4_v[c, pl.ds(lo, 16)] = h4
            idx5_v[c, pl.ds(lo, 16)] = h5
            return _

        lax.fori_loop(jnp.int32(0), jnp.int32(_GRP), hash_body, None)

        for c in range(_NCHUNK):
            ci = jnp.int32(c)
            cp3 = pltpu.async_copy(t3_hbm.at[idx3_v.at[ci]], r3_v, sem)
            cp4 = pltpu.async_copy(t4_hbm.at[idx4_v.at[ci]], r4_v, sem)
            cp5 = pltpu.async_copy(t5_hbm.at[idx5_v.at[ci]], r5_v, sem)
            cp3.wait()
            cp4.wait()
            cp5.wait()

            def add_body(r, _):
                for q in range(_HD // 16):
                    sl = pl.ds(q * 16, 16)
                    r3_v[r, sl] = r3_v[r, sl] + r4_v[r, sl] + r5_v[r, sl]
                return _

            lax.fori_loop(jnp.int32(0), jnp.int32(_CHUNK), add_body, None)
            pltpu.sync_copy(
                r3_v, out_hbm.at[pl.ds(wid * _TPW + c * _CHUNK, _CHUNK)])

    return body(ids_pad, ht3, ht4, ht5)


_BLK = 512
_NBLK = (_B * _S) // _BLK


def _tc_body(ids_ref, hsum_ref, pos_ref, byte_ref, pw_ref, pb_ref,
             g_ref, be_ref, out_ref):
    ids = ids_ref[0, 0, :]
    onehot = (ids[:, None]
              == lax.broadcasted_iota(jnp.int32, (_BLK, _VOCAB), 1)
              ).astype(jnp.float32)
    x = lax.dot(onehot, byte_ref[...], preferred_element_type=jnp.float32)
    x = x + pos_ref[...]
    x = x + lax.dot(hsum_ref[...], pw_ref[...],
                    preferred_element_type=jnp.float32)
    x = x + pb_ref[...]
    mean = jnp.mean(x, axis=1, keepdims=True)
    xc = x - mean
    var = jnp.mean(xc * xc, axis=1, keepdims=True)
    inv = lax.rsqrt(var + 1e-12)
    out_ref[...] = xc * inv * g_ref[...] + be_ref[...]


def _tc_combine(ids3d, hsum, pos_table, byte_table, proj_W, proj_b,
                gamma, beta):
    sblocks = _S // _BLK
    z = lambda: jnp.int32(0)
    return pl.pallas_call(
        _tc_body,
        grid=(_B, sblocks),
        in_specs=[
            pl.BlockSpec((1, 1, _BLK), lambda b, j: (b * sblocks + j, z(), z())),
            pl.BlockSpec((_BLK, _HD), lambda b, j: (b * sblocks + j, z())),
            pl.BlockSpec((_BLK, _H), lambda b, j: (j, z())),
            pl.BlockSpec((_VOCAB, _H), lambda b, j: (z(), z())),
            pl.BlockSpec((_HD, _H), lambda b, j: (z(), z())),
            pl.BlockSpec((1, _H), lambda b, j: (z(), z())),
            pl.BlockSpec((1, _H), lambda b, j: (z(), z())),
            pl.BlockSpec((1, _H), lambda b, j: (z(), z())),
        ],
        out_specs=pl.BlockSpec((_BLK, _H), lambda b, j: (b * sblocks + j, z())),
        out_shape=jax.ShapeDtypeStruct((_B * _S, _H), jnp.float32),
    )(ids3d, hsum, pos_table, byte_table, proj_W, proj_b, gamma, beta)


def kernel(input_ids, byte_table, pos_table, ht3, ht4, ht5, proj_W, proj_b,
           gamma, beta):
    ids32 = input_ids.astype(jnp.int32)
    ids_pad = jnp.pad(ids32, ((0, 0), (_PAD, 0))).reshape(-1)
    hsum = _sc_hash_gather(ids_pad, ht3, ht4, ht5)
    ids3d = ids32.reshape(_NBLK, 1, _BLK)
    out = _tc_combine(ids3d, hsum, pos_table, byte_table,
                      proj_W.astype(jnp.float32),
                      proj_b.reshape(1, _H), gamma.reshape(1, _H),
                      beta.reshape(1, _H))
    return out.reshape(_B, _S, _H).astype(jnp.float64)


# speedup vs baseline: 6.9824x; 1.0120x over previous
import functools

import jax
import jax.numpy as jnp
from jax import lax
from jax.experimental import pallas as pl
from jax.experimental.pallas import tpu as pltpu
from jax.experimental.pallas import tpu_sc as plsc

_B, _S, _H = 4, 8192, 768
_VOCAB = 260
_HASH_VOCAB = 100000
_HD = 128

_P1 = 257
_P2 = 66049
_P3 = 74593
_P4 = 70401

_NC, _NS = 2, 16
_NW = _NC * _NS
_TPW = (_B * _S) // _NW
_PAD = 8
_CHUNK = 128
_NCHUNK = _TPW // _CHUNK
_GRP = _TPW // 16


def _sc_hash_gather(ids_pad, ht3, ht4, ht5):
    mesh = plsc.VectorSubcoreMesh(core_axis_name="c", subcore_axis_name="s",
                                  num_cores=_NC, num_subcores=_NS)

    @functools.partial(
        pl.kernel,
        out_type=jax.ShapeDtypeStruct((_B * _S, _HD), jnp.float32),
        mesh=mesh,
        scratch_types=[
            pltpu.VMEM((_TPW + _PAD,), jnp.int32),
            pltpu.VMEM((_NCHUNK, _CHUNK), jnp.int32),
            pltpu.VMEM((_NCHUNK, _CHUNK), jnp.int32),
            pltpu.VMEM((_NCHUNK, _CHUNK), jnp.int32),
            pltpu.VMEM((_CHUNK, _HD), jnp.float32),
            pltpu.VMEM((_CHUNK, _HD), jnp.float32),
            pltpu.VMEM((_CHUNK, _HD), jnp.float32),
            pltpu.SemaphoreType.DMA,
        ],
    )
    def body(ids_hbm, t3_hbm, t4_hbm, t5_hbm, out_hbm,
             ids_v, idx3_v, idx4_v, idx5_v, r3_v, r4_v, r5_v, sem):
        wid = lax.axis_index("s") * _NC + lax.axis_index("c")
        wpr = jnp.int32(_S // _TPW)
        b = lax.div(wid, wpr)
        base_in_row = lax.rem(wid, wpr) * _TPW
        start = b * (_S + _PAD) + base_in_row

        pltpu.sync_copy(ids_hbm.at[pl.ds(start, _TPW + _PAD)], ids_v)

        def hash_body(g, _):
            o = g * 16
            vj = ids_v[pl.ds(_PAD + o, 16)]
            v1 = ids_v[pl.ds(_PAD - 1 + o, 16)]
            v2 = ids_v[pl.ds(_PAD - 2 + o, 16)]
            v3 = ids_v[pl.ds(_PAD - 3 + o, 16)]
            v4 = ids_v[pl.ds(_PAD - 4 + o, 16)]
            hv = jnp.int32(_HASH_VOCAB)
            h3 = lax.rem(v2 * _P2 + v1 * _P1 + vj, hv)
            h4 = lax.rem(v3 * _P3 + h3, hv)
            h5 = lax.rem(v4 * _P4 + h4, hv)
            gpc = jnp.int32(_CHUNK // 16)
            c = lax.div(g, gpc)
            lo = lax.rem(g, gpc) * 16
            idx3_v[c, pl.ds(lo, 16)] = h3
            idx4_v[c, pl.ds(lo, 16)] = h4
            idx5_v[c, pl.ds(lo, 16)] = h5
            return _

        lax.fori_loop(jnp.int32(0), jnp.int32(_GRP), hash_body, None)

        for c in range(_NCHUNK):
            ci = jnp.int32(c)
            cp3 = pltpu.async_copy(t3_hbm.at[idx3_v.at[ci]], r3_v, sem)
            cp4 = pltpu.async_copy(t4_hbm.at[idx4_v.at[ci]], r4_v, sem)
            cp5 = pltpu.async_copy(t5_hbm.at[idx5_v.at[ci]], r5_v, sem)
            cp3.wait()
            cp4.wait()
            cp5.wait()

            def add_body(r, _):
                for q in range(_HD // 16):
                    sl = pl.ds(q * 16, 16)
                    r3_v[r, sl] = r3_v[r, sl] + r4_v[r, sl] + r5_v[r, sl]
                return _

            lax.fori_loop(jnp.int32(0), jnp.int32(_CHUNK), add_body, None)
            pltpu.sync_copy(
                r3_v, out_hbm.at[pl.ds(wid * _TPW + c * _CHUNK, _CHUNK)])

    return body(ids_pad, ht3, ht4, ht5)


_BLK = 1024
_NBLK = (_B * _S) // _BLK


def _tc_body(ids_ref, hsum_ref, pos_ref, byte_ref, pw_ref, pb_ref,
             g_ref, be_ref, out_ref):
    ids = ids_ref[0, 0, :]
    onehot = (ids[:, None]
              == lax.broadcasted_iota(jnp.int32, (_BLK, _VOCAB), 1)
              ).astype(jnp.bfloat16)
    x = lax.dot(onehot, byte_ref[...], preferred_element_type=jnp.float32)
    x = x + pos_ref[...]
    x = x + lax.dot(hsum_ref[...], pw_ref[...],
                    preferred_element_type=jnp.float32)
    x = x + pb_ref[...]
    mean = jnp.mean(x, axis=1, keepdims=True)
    xc = x - mean
    var = jnp.mean(xc * xc, axis=1, keepdims=True)
    inv = lax.rsqrt(var + 1e-12)
    out_ref[...] = xc * inv * g_ref[...] + be_ref[...]


def _tc_combine(ids3d, hsum, pos_table, byte_table, proj_W, proj_b,
                gamma, beta):
    sblocks = _S // _BLK
    z = lambda: jnp.int32(0)
    return pl.pallas_call(
        _tc_body,
        grid=(_B, sblocks),
        in_specs=[
            pl.BlockSpec((1, 1, _BLK), lambda b, j: (b * sblocks + j, z(), z())),
            pl.BlockSpec((_BLK, _HD), lambda b, j: (b * sblocks + j, z())),
            pl.BlockSpec((_BLK, _H), lambda b, j: (j, z())),
            pl.BlockSpec((_VOCAB, _H), lambda b, j: (z(), z())),
            pl.BlockSpec((_HD, _H), lambda b, j: (z(), z())),
            pl.BlockSpec((1, _H), lambda b, j: (z(), z())),
            pl.BlockSpec((1, _H), lambda b, j: (z(), z())),
            pl.BlockSpec((1, _H), lambda b, j: (z(), z())),
        ],
        out_specs=pl.BlockSpec((_BLK, _H), lambda b, j: (b * sblocks + j, z())),
        out_shape=jax.ShapeDtypeStruct((_B * _S, _H), jnp.float32),
    )(ids3d, hsum, pos_table, byte_table, proj_W, proj_b, gamma, beta)


def kernel(input_ids, byte_table, pos_table, ht3, ht4, ht5, proj_W, proj_b,
           gamma, beta):
    ids32 = input_ids.astype(jnp.int32)
    ids_pad = jnp.pad(ids32, ((0, 0), (_PAD, 0))).reshape(-1)
    hsum = _sc_hash_gather(ids_pad, ht3, ht4, ht5)
    ids3d = ids32.reshape(_NBLK, 1, _BLK)
    out = _tc_combine(ids3d, hsum, pos_table, byte_table.astype(jnp.bfloat16),
                      proj_W.astype(jnp.float32),
                      proj_b.reshape(1, _H), gamma.reshape(1, _H),
                      beta.reshape(1, _H))
    return out.reshape(_B, _S, _H).astype(jnp.float64)


# speedup vs baseline: 7.0448x; 1.0089x over previous
import functools

import jax
import jax.numpy as jnp
from jax import lax
from jax.experimental import pallas as pl
from jax.experimental.pallas import tpu as pltpu
from jax.experimental.pallas import tpu_sc as plsc

_B, _S, _H = 4, 8192, 768
_VOCAB = 260
_HASH_VOCAB = 100000
_HD = 128

_P1 = 257
_P2 = 66049
_P3 = 74593
_P4 = 70401

_NC, _NS = 2, 16
_NW = _NC * _NS
_HB = _B // 2
_HTOK = _HB * _S
_TPW = _HTOK // _NW
_PAD = 8
_CHUNK = 128
_NCHUNK = _TPW // _CHUNK
_GRP = _TPW // 16
_ROWLEN = _S + _PAD


def _sc_hash_gather(ids_pad_half, ht3, ht4, ht5):
    mesh = plsc.VectorSubcoreMesh(core_axis_name="c", subcore_axis_name="s",
                                  num_cores=_NC, num_subcores=_NS)

    @functools.partial(
        pl.kernel,
        out_type=jax.ShapeDtypeStruct((_HTOK, _HD), jnp.float32),
        mesh=mesh,
        scratch_types=[
            pltpu.VMEM((_TPW + _PAD,), jnp.int32),
            pltpu.VMEM((_NCHUNK, _CHUNK), jnp.int32),
            pltpu.VMEM((_NCHUNK, _CHUNK), jnp.int32),
            pltpu.VMEM((_NCHUNK, _CHUNK), jnp.int32),
            pltpu.VMEM((_CHUNK, _HD), jnp.float32),
            pltpu.VMEM((_CHUNK, _HD), jnp.float32),
            pltpu.VMEM((_CHUNK, _HD), jnp.float32),
            pltpu.SemaphoreType.DMA,
        ],
    )
    def body(ids_hbm, t3_hbm, t4_hbm, t5_hbm, out_hbm,
             ids_v, idx3_v, idx4_v, idx5_v, r3_v, r4_v, r5_v, sem):
        wid = lax.axis_index("s") * _NC + lax.axis_index("c")
        wpr = jnp.int32(_S // _TPW)
        b = lax.div(wid, wpr)
        base_in_row = lax.rem(wid, wpr) * _TPW
        start = b * _ROWLEN + base_in_row

        pltpu.sync_copy(ids_hbm.at[pl.ds(start, _TPW + _PAD)], ids_v)

        def hash_body(g, _):
            o = g * 16
            vj = ids_v[pl.ds(_PAD + o, 16)]
            v1 = ids_v[pl.ds(_PAD - 1 + o, 16)]
            v2 = ids_v[pl.ds(_PAD - 2 + o, 16)]
            v3 = ids_v[pl.ds(_PAD - 3 + o, 16)]
            v4 = ids_v[pl.ds(_PAD - 4 + o, 16)]
            hv = jnp.int32(_HASH_VOCAB)
            h3 = lax.rem(v2 * _P2 + v1 * _P1 + vj, hv)
            h4 = lax.rem(v3 * _P3 + h3, hv)
            h5 = lax.rem(v4 * _P4 + h4, hv)
            gpc = jnp.int32(_CHUNK // 16)
            c = lax.div(g, gpc)
            lo = lax.rem(g, gpc) * 16
            idx3_v[c, pl.ds(lo, 16)] = h3
            idx4_v[c, pl.ds(lo, 16)] = h4
            idx5_v[c, pl.ds(lo, 16)] = h5
            return _

        lax.fori_loop(jnp.int32(0), jnp.int32(_GRP), hash_body, None)

        for c in range(_NCHUNK):
            ci = jnp.int32(c)
            cp3 = pltpu.async_copy(t3_hbm.at[idx3_v.at[ci]], r3_v, sem)
            cp4 = pltpu.async_copy(t4_hbm.at[idx4_v.at[ci]], r4_v, sem)
            cp5 = pltpu.async_copy(t5_hbm.at[idx5_v.at[ci]], r5_v, sem)
            cp3.wait()
            cp4.wait()
            cp5.wait()

            def add_body(r, _):
                for q in range(_HD // 16):
                    sl = pl.ds(q * 16, 16)
                    r3_v[r, sl] = r3_v[r, sl] + r4_v[r, sl] + r5_v[r, sl]
                return _

            lax.fori_loop(jnp.int32(0), jnp.int32(_CHUNK), add_body, None)
            pltpu.sync_copy(
                r3_v, out_hbm.at[pl.ds(wid * _TPW + c * _CHUNK, _CHUNK)])

    return body(ids_pad_half, ht3, ht4, ht5)


_BLK = 1024
_SBLK = _S // _BLK
_HBLK = _HB * _SBLK


def _tc_body(ids_ref, hsum_ref, pos_ref, byte_ref, pw_ref, pb_ref,
             g_ref, be_ref, *rest):
    out_ref = rest[-1]
    ids = ids_ref[0, 0, :]
    onehot = (ids[:, None]
              == lax.broadcasted_iota(jnp.int32, (_BLK, _VOCAB), 1)
              ).astype(jnp.bfloat16)
    x = lax.dot(onehot, byte_ref[...], preferred_element_type=jnp.float32)
    x = x + pos_ref[...]
    x = x + lax.dot(hsum_ref[...], pw_ref[...],
                    preferred_element_type=jnp.float32)
    x = x + pb_ref[...]
    mean = jnp.mean(x, axis=1, keepdims=True)
    xc = x - mean
    var = jnp.mean(xc * xc, axis=1, keepdims=True)
    inv = lax.rsqrt(var + 1e-12)
    out_ref[...] = xc * inv * g_ref[...] + be_ref[...]


def _tc_combine(half, ids3d, hsum, pos_table, byte_table, proj_W, proj_b,
                gamma, beta, prev_buf):
    z = lambda: jnp.int32(0)
    off = half * _HBLK
    in_specs = [
        pl.BlockSpec((1, 1, _BLK), lambda b, j: (b * _SBLK + j, z(), z())),
        pl.BlockSpec((_BLK, _HD), lambda b, j: (b * _SBLK + j, z())),
        pl.BlockSpec((_BLK, _H), lambda b, j: (j, z())),
        pl.BlockSpec((_VOCAB, _H), lambda b, j: (z(), z())),
        pl.BlockSpec((_HD, _H), lambda b, j: (z(), z())),
        pl.BlockSpec((1, _H), lambda b, j: (z(), z())),
        pl.BlockSpec((1, _H), lambda b, j: (z(), z())),
        pl.BlockSpec((1, _H), lambda b, j: (z(), z())),
    ]
    args = [ids3d, hsum, pos_table, byte_table, proj_W, proj_b, gamma, beta]
    aliases = {}
    if prev_buf is not None:
        in_specs.append(pl.BlockSpec(memory_space=pl.ANY))
        args.append(prev_buf)
        aliases = {8: 0}
    return pl.pallas_call(
        _tc_body,
        grid=(_HB, _SBLK),
        in_specs=in_specs,
        out_specs=pl.BlockSpec(
            (_BLK, _H), lambda b, j: (b * _SBLK + j + off, z())),
        out_shape=jax.ShapeDtypeStruct((_B * _S, _H), jnp.float32),
        input_output_aliases=aliases,
    )(*args)


def kernel(input_ids, byte_table, pos_table, ht3, ht4, ht5, proj_W, proj_b,
           gamma, beta):
    ids32 = input_ids.astype(jnp.int32)
    ids_pad = jnp.pad(ids32, ((0, 0), (_PAD, 0))).reshape(-1)
    hsum0 = _sc_hash_gather(ids_pad[: _HB * _ROWLEN], ht3, ht4, ht5)
    hsum1 = _sc_hash_gather(ids_pad[_HB * _ROWLEN:], ht3, ht4, ht5)
    byte16 = byte_table.astype(jnp.bfloat16)
    pw32 = proj_W.astype(jnp.float32)
    pb = proj_b.reshape(1, _H)
    ga = gamma.reshape(1, _H)
    be = beta.reshape(1, _H)
    ids3d0 = ids32[:_HB].reshape(_HBLK, 1, _BLK)
    ids3d1 = ids32[_HB:].reshape(_HBLK, 1, _BLK)
    buf = _tc_combine(0, ids3d0, hsum0, pos_table, byte16, pw32, pb, ga, be,
                      None)
    out = _tc_combine(1, ids3d1, hsum1, pos_table, byte16, pw32, pb, ga, be,
                      buf)
    return out.reshape(_B, _S, _H).astype(jnp.float64)


# speedup vs baseline: 7.0561x; 1.0016x over previous
import functools

import jax
import jax.numpy as jnp
from jax import lax
from jax.experimental import pallas as pl
from jax.experimental.pallas import tpu as pltpu
from jax.experimental.pallas import tpu_sc as plsc

_B, _S, _H = 4, 8192, 768
_VOCAB = 260
_HASH_VOCAB = 100000
_HD = 128

_P1 = 257
_P2 = 66049
_P3 = 74593
_P4 = 70401

_NC, _NS = 2, 16
_NW = _NC * _NS
_HB = _B // 2
_HTOK = _HB * _S
_TPW = _HTOK // _NW
_PAD = 8
_CHUNK = 128
_NCHUNK = _TPW // _CHUNK
_GRP = _TPW // 16
_ROWLEN = _S + _PAD


def _sc_hash_gather(ids_pad_half, ht3, ht4, ht5):
    mesh = plsc.VectorSubcoreMesh(core_axis_name="c", subcore_axis_name="s",
                                  num_cores=_NC, num_subcores=_NS)

    @functools.partial(
        pl.kernel,
        out_type=jax.ShapeDtypeStruct((_HTOK, _HD), jnp.float32),
        mesh=mesh,
        scratch_types=[
            pltpu.VMEM((_TPW + _PAD,), jnp.int32),
            pltpu.VMEM((_NCHUNK, _CHUNK), jnp.int32),
            pltpu.VMEM((_NCHUNK, _CHUNK), jnp.int32),
            pltpu.VMEM((_NCHUNK, _CHUNK), jnp.int32),
            pltpu.VMEM((_CHUNK, _HD), jnp.float32),
            pltpu.VMEM((_CHUNK, _HD), jnp.float32),
            pltpu.VMEM((_CHUNK, _HD), jnp.float32),
            pltpu.SemaphoreType.DMA,
        ],
    )
    def body(ids_hbm, t3_hbm, t4_hbm, t5_hbm, out_hbm,
             ids_v, idx3_v, idx4_v, idx5_v, r3_v, r4_v, r5_v, sem):
        wid = lax.axis_index("s") * _NC + lax.axis_index("c")
        wpr = jnp.int32(_S // _TPW)
        b = lax.div(wid, wpr)
        base_in_row = lax.rem(wid, wpr) * _TPW
        start = b * _ROWLEN + base_in_row

        pltpu.sync_copy(ids_hbm.at[pl.ds(start, _TPW + _PAD)], ids_v)

        def hash_body(g, _):
            o = g * 16
            vj = ids_v[pl.ds(_PAD + o, 16)]
            v1 = ids_v[pl.ds(_PAD - 1 + o, 16)]
            v2 = ids_v[pl.ds(_PAD - 2 + o, 16)]
            v3 = ids_v[pl.ds(_PAD - 3 + o, 16)]
            v4 = ids_v[pl.ds(_PAD - 4 + o, 16)]
            hv = jnp.int32(_HASH_VOCAB)
            h3 = lax.rem(v2 * _P2 + v1 * _P1 + vj, hv)
            h4 = lax.rem(v3 * _P3 + h3, hv)
            h5 = lax.rem(v4 * _P4 + h4, hv)
            gpc = jnp.int32(_CHUNK // 16)
            c = lax.div(g, gpc)
            lo = lax.rem(g, gpc) * 16
            idx3_v[c, pl.ds(lo, 16)] = h3
            idx4_v[c, pl.ds(lo, 16)] = h4
            idx5_v[c, pl.ds(lo, 16)] = h5
            return _

        lax.fori_loop(jnp.int32(0), jnp.int32(_GRP), hash_body, None)

        for c in range(_NCHUNK):
            ci = jnp.int32(c)
            cp3 = pltpu.async_copy(t3_hbm.at[idx3_v.at[ci]], r3_v, sem)
            cp4 = pltpu.async_copy(t4_hbm.at[idx4_v.at[ci]], r4_v, sem)
            cp5 = pltpu.async_copy(t5_hbm.at[idx5_v.at[ci]], r5_v, sem)
            cp3.wait()
            cp4.wait()
            cp5.wait()

            def add_body(r, _):
                for q in range(_HD // 16):
                    sl = pl.ds(q * 16, 16)
                    r3_v[r, sl] = r3_v[r, sl] + r4_v[r, sl] + r5_v[r, sl]
                return _

            lax.fori_loop(jnp.int32(0), jnp.int32(_CHUNK), add_body, None)
            pltpu.sync_copy(
                r3_v, out_hbm.at[pl.ds(wid * _TPW + c * _CHUNK, _CHUNK)])

    return body(ids_pad_half, ht3, ht4, ht5)


_BLK = 1024
_SBLK = _S // _BLK
_HBLK = _HB * _SBLK


def _tc_body(ids_ref, hsum_ref, pos_ref, byte_ref, pw_ref, pb_ref,
             g_ref, be_ref, *rest):
    out_ref = rest[-1]
    ids = ids_ref[0, 0, :]
    onehot = (ids[:, None]
              == lax.broadcasted_iota(jnp.int32, (_BLK, _VOCAB), 1)
              ).astype(jnp.bfloat16)
    x = lax.dot(onehot, byte_ref[...], preferred_element_type=jnp.float32)
    x = x + pos_ref[...]
    x = x + lax.dot(hsum_ref[...], pw_ref[...],
                    preferred_element_type=jnp.float32)
    x = x + pb_ref[...]
    mean = jnp.mean(x, axis=1, keepdims=True)
    xc = x - mean
    var = jnp.mean(xc * xc, axis=1, keepdims=True)
    inv = lax.rsqrt(var + 1e-12)
    out_ref[...] = xc * inv * g_ref[...] + be_ref[...]


def _tc_combine(half, ids3d, hsum, pos_table, byte_table, proj_W, proj_b,
                gamma, beta, prev_buf):
    z = lambda: jnp.int32(0)
    off = half * _HBLK
    in_specs = [
        pl.BlockSpec((1, 1, _BLK), lambda j, b: (b * _SBLK + j, z(), z())),
        pl.BlockSpec((_BLK, _HD), lambda j, b: (b * _SBLK + j, z())),
        pl.BlockSpec((_BLK, _H), lambda j, b: (j, z())),
        pl.BlockSpec((_VOCAB, _H), lambda j, b: (z(), z())),
        pl.BlockSpec((_HD, _H), lambda j, b: (z(), z())),
        pl.BlockSpec((1, _H), lambda j, b: (z(), z())),
        pl.BlockSpec((1, _H), lambda j, b: (z(), z())),
        pl.BlockSpec((1, _H), lambda j, b: (z(), z())),
    ]
    args = [ids3d, hsum, pos_table, byte_table, proj_W, proj_b, gamma, beta]
    aliases = {}
    if prev_buf is not None:
        in_specs.append(pl.BlockSpec(memory_space=pl.ANY))
        args.append(prev_buf)
        aliases = {8: 0}
    return pl.pallas_call(
        _tc_body,
        grid=(_SBLK, _HB),
        in_specs=in_specs,
        out_specs=pl.BlockSpec(
            (_BLK, _H), lambda j, b: (b * _SBLK + j + off, z())),
        out_shape=jax.ShapeDtypeStruct((_B * _S, _H), jnp.float32),
        input_output_aliases=aliases,
    )(*args)


def kernel(input_ids, byte_table, pos_table, ht3, ht4, ht5, proj_W, proj_b,
           gamma, beta):
    ids32 = input_ids.astype(jnp.int32)
    ids_pad = jnp.pad(ids32, ((0, 0), (_PAD, 0))).reshape(-1)
    hsum0 = _sc_hash_gather(ids_pad[: _HB * _ROWLEN], ht3, ht4, ht5)
    hsum1 = _sc_hash_gather(ids_pad[_HB * _ROWLEN:], ht3, ht4, ht5)
    byte16 = byte_table.astype(jnp.bfloat16)
    pw32 = proj_W.astype(jnp.float32)
    pb = proj_b.reshape(1, _H)
    ga = gamma.reshape(1, _H)
    be = beta.reshape(1, _H)
    ids3d0 = ids32[:_HB].reshape(_HBLK, 1, _BLK)
    ids3d1 = ids32[_HB:].reshape(_HBLK, 1, _BLK)
    buf = _tc_combine(0, ids3d0, hsum0, pos_table, byte16, pw32, pb, ga, be,
                      None)
    out = _tc_combine(1, ids3d1, hsum1, pos_table, byte16, pw32, pb, ga, be,
                      buf)
    return out.reshape(_B, _S, _H).astype(jnp.float64)


# speedup vs baseline: 7.0625x; 1.0009x over previous
import functools

import jax
import jax.numpy as jnp
from jax import lax
from jax.experimental import pallas as pl
from jax.experimental.pallas import tpu as pltpu
from jax.experimental.pallas import tpu_sc as plsc

_B, _S, _H = 4, 8192, 768
_VOCAB = 260
_HASH_VOCAB = 100000
_HD = 128

_P1 = 257
_P2 = 66049
_P3 = 74593
_P4 = 70401

_NC, _NS = 2, 16
_NW = _NC * _NS
_HB = _B // 2
_HTOK = _HB * _S
_TPW = _HTOK // _NW
_PAD = 8
_CHUNK = 128
_NCHUNK = _TPW // _CHUNK
_GRP = _TPW // 16
_ROWLEN = _S + _PAD


def _sc_hash_gather(ids_pad_half, ht3, ht4, ht5):
    mesh = plsc.VectorSubcoreMesh(core_axis_name="c", subcore_axis_name="s",
                                  num_cores=_NC, num_subcores=_NS)

    @functools.partial(
        pl.kernel,
        out_type=jax.ShapeDtypeStruct((_HTOK, _HD), jnp.float32),
        mesh=mesh,
        scratch_types=[
            pltpu.VMEM((_TPW + _PAD,), jnp.int32),
            pltpu.VMEM((_NCHUNK, _CHUNK), jnp.int32),
            pltpu.VMEM((_NCHUNK, _CHUNK), jnp.int32),
            pltpu.VMEM((_NCHUNK, _CHUNK), jnp.int32),
            pltpu.VMEM((2, _CHUNK, _HD), jnp.float32),
            pltpu.VMEM((2, _CHUNK, _HD), jnp.float32),
            pltpu.VMEM((2, _CHUNK, _HD), jnp.float32),
            pltpu.SemaphoreType.DMA,
            pltpu.SemaphoreType.DMA,
            pltpu.SemaphoreType.DMA,
            pltpu.SemaphoreType.DMA,
        ],
    )
    def body(ids_hbm, t3_hbm, t4_hbm, t5_hbm, out_hbm,
             ids_v, idx3_v, idx4_v, idx5_v, r3_v, r4_v, r5_v,
             gsem0, gsem1, ssem0, ssem1):
        wid = lax.axis_index("s") * _NC + lax.axis_index("c")
        wpr = jnp.int32(_S // _TPW)
        b = lax.div(wid, wpr)
        base_in_row = lax.rem(wid, wpr) * _TPW
        start = b * _ROWLEN + base_in_row

        pltpu.sync_copy(ids_hbm.at[pl.ds(start, _TPW + _PAD)], ids_v)

        def hash_body(g, _):
            o = g * 16
            vj = ids_v[pl.ds(_PAD + o, 16)]
            v1 = ids_v[pl.ds(_PAD - 1 + o, 16)]
            v2 = ids_v[pl.ds(_PAD - 2 + o, 16)]
            v3 = ids_v[pl.ds(_PAD - 3 + o, 16)]
            v4 = ids_v[pl.ds(_PAD - 4 + o, 16)]
            hv = jnp.int32(_HASH_VOCAB)
            h3 = lax.rem(v2 * _P2 + v1 * _P1 + vj, hv)
            h4 = lax.rem(v3 * _P3 + h3, hv)
            h5 = lax.rem(v4 * _P4 + h4, hv)
            gpc = jnp.int32(_CHUNK // 16)
            c = lax.div(g, gpc)
            lo = lax.rem(g, gpc) * 16
            idx3_v[c, pl.ds(lo, 16)] = h3
            idx4_v[c, pl.ds(lo, 16)] = h4
            idx5_v[c, pl.ds(lo, 16)] = h5
            return _

        lax.fori_loop(jnp.int32(0), jnp.int32(_GRP), hash_body, None)

        gsems = (gsem0, gsem1)
        ssems = (ssem0, ssem1)

        def fire(c):
            s = c % 2
            ci = jnp.int32(c)
            si = jnp.int32(s)
            g = (pltpu.async_copy(t3_hbm.at[idx3_v.at[ci]], r3_v.at[si],
                                  gsems[s]),
                 pltpu.async_copy(t4_hbm.at[idx4_v.at[ci]], r4_v.at[si],
                                  gsems[s]),
                 pltpu.async_copy(t5_hbm.at[idx5_v.at[ci]], r5_v.at[si],
                                  gsems[s]))
            return g

        gd = {0: fire(0)}
        st = {}
        for c in range(_NCHUNK):
            s = c % 2
            si = jnp.int32(s)
            if c >= 1:
                st[c - 1].wait()
            if c + 1 < _NCHUNK:
                gd[c + 1] = fire(c + 1)
            for g in gd[c]:
                g.wait()

            def add_body(r, _):
                for q in range(_HD // 16):
                    sl = pl.ds(q * 16, 16)
                    r3_v[si, r, sl] = (r3_v[si, r, sl] + r4_v[si, r, sl]
                                       + r5_v[si, r, sl])
                return _

            lax.fori_loop(jnp.int32(0), jnp.int32(_CHUNK), add_body, None)
            st[c] = pltpu.async_copy(
                r3_v.at[si],
                out_hbm.at[pl.ds(wid * _TPW + c * _CHUNK, _CHUNK)],
                ssems[s])
        st[_NCHUNK - 1].wait()

    return body(ids_pad_half, ht3, ht4, ht5)


_BLK = 1024
_SBLK = _S // _BLK
_HBLK = _HB * _SBLK


def _tc_body(ids_ref, hsum_ref, pos_ref, byte_ref, pw_ref, pb_ref,
             g_ref, be_ref, *rest):
    out_ref = rest[-1]
    ids = ids_ref[0, 0, :]
    onehot = (ids[:, None]
              == lax.broadcasted_iota(jnp.int32, (_BLK, _VOCAB), 1)
              ).astype(jnp.bfloat16)
    x = lax.dot(onehot, byte_ref[...], preferred_element_type=jnp.float32)
    x = x + pos_ref[...]
    x = x + lax.dot(hsum_ref[...], pw_ref[...],
                    preferred_element_type=jnp.float32)
    x = x + pb_ref[...]
    mean = jnp.mean(x, axis=1, keepdims=True)
    xc = x - mean
    var = jnp.mean(xc * xc, axis=1, keepdims=True)
    inv = lax.rsqrt(var + 1e-12)
    out_ref[...] = xc * inv * g_ref[...] + be_ref[...]


def _tc_combine(half, ids3d, hsum, pos_table, byte_table, proj_W, proj_b,
                gamma, beta, prev_buf):
    z = lambda: jnp.int32(0)
    off = half * _HBLK
    in_specs = [
        pl.BlockSpec((1, 1, _BLK), lambda j, b: (b * _SBLK + j, z(), z())),
        pl.BlockSpec((_BLK, _HD), lambda j, b: (b * _SBLK + j, z())),
        pl.BlockSpec((_BLK, _H), lambda j, b: (j, z())),
        pl.BlockSpec((_VOCAB, _H), lambda j, b: (z(), z())),
        pl.BlockSpec((_HD, _H), lambda j, b: (z(), z())),
        pl.BlockSpec((1, _H), lambda j, b: (z(), z())),
        pl.BlockSpec((1, _H), lambda j, b: (z(), z())),
        pl.BlockSpec((1, _H), lambda j, b: (z(), z())),
    ]
    args = [ids3d, hsum, pos_table, byte_table, proj_W, proj_b, gamma, beta]
    aliases = {}
    if prev_buf is not None:
        in_specs.append(pl.BlockSpec(memory_space=pl.ANY))
        args.append(prev_buf)
        aliases = {8: 0}
    return pl.pallas_call(
        _tc_body,
        grid=(_SBLK, _HB),
        in_specs=in_specs,
        out_specs=pl.BlockSpec(
            (_BLK, _H), lambda j, b: (b * _SBLK + j + off, z())),
        out_shape=jax.ShapeDtypeStruct((_B * _S, _H), jnp.float32),
        input_output_aliases=aliases,
    )(*args)


def kernel(input_ids, byte_table, pos_table, ht3, ht4, ht5, proj_W, proj_b,
           gamma, beta):
    ids32 = input_ids.astype(jnp.int32)
    ids_pad = jnp.pad(ids32, ((0, 0), (_PAD, 0))).reshape(-1)
    hsum0 = _sc_hash_gather(ids_pad[: _HB * _ROWLEN], ht3, ht4, ht5)
    hsum1 = _sc_hash_gather(ids_pad[_HB * _ROWLEN:], ht3, ht4, ht5)
    byte16 = byte_table.astype(jnp.bfloat16)
    pw32 = proj_W.astype(jnp.float32)
    pb = proj_b.reshape(1, _H)
    ga = gamma.reshape(1, _H)
    be = beta.reshape(1, _H)
    ids3d0 = ids32[:_HB].reshape(_HBLK, 1, _BLK)
    ids3d1 = ids32[_HB:].reshape(_HBLK, 1, _BLK)
    buf = _tc_combine(0, ids3d0, hsum0, pos_table, byte16, pw32, pb, ga, be,
                      None)
    out = _tc_combine(1, ids3d1, hsum1, pos_table, byte16, pw32, pb, ga, be,
                      buf)
    return out.reshape(_B, _S, _H).astype(jnp.float64)
